# Initial kernel scaffold; baseline (speedup 1.0000x reference)
import jax
import jax.numpy as jnp
from jax.experimental import pallas as pl


def kernel(input_ids, emb_table, fc_w, fc_b):
    raise NotImplementedError("write your pallas kernel here")


# speedup vs baseline: 1.5977x; 1.5977x over previous
import functools

import jax
import jax.numpy as jnp
from jax import lax
from jax.experimental import pallas as pl
from jax.experimental.pallas import tpu as pltpu
from jax.experimental.pallas import tpu_sc as plsc

_VOCAB = 100000
_D = 128
_B = 4096
_H = 200

_NC = 2
_NS = 16
_NW = _NC * _NS
_LANES = 16
_NV = _D // _LANES

_G = 100
_GROUPS_PER_ROW = _H // _G
_ROWS_PER_W = _B // _NW
_GROUPS_PER_W = _ROWS_PER_W * _GROUPS_PER_ROW
_BLK_GROUPS = 32
_NBLK = _GROUPS_PER_W // _BLK_GROUPS
_ROWS_PER_BLK = _BLK_GROUPS // _GROUPS_PER_ROW
_NBUF = 4
_SCALE = 1.0 / _H


def _pool_body(ids_hbm, table_hbm, out_hbm, ids_v, out_v,
               buf0, buf1, buf2, buf3, sem0, sem1, sem2, sem3):
    bufs = (buf0, buf1, buf2, buf3)
    sems = (sem0, sem1, sem2, sem3)
    wid = lax.axis_index("s") * _NC + lax.axis_index("c")
    gbase = wid * _GROUPS_PER_W

    def fire(g, b):
        pltpu.make_async_copy(table_hbm.at[ids_v.at[g]], bufs[b], sems[b]).start()

    def drain(g, b):
        pltpu.make_async_copy(table_hbm.at[ids_v.at[g]], bufs[b], sems[b]).wait()

    def accum(buf, init):
        def body(r, acc):
            return tuple(acc[c] + buf[r, pl.ds(c * _LANES, _LANES)]
                         for c in range(_NV))
        return lax.fori_loop(0, _G, body, init)

    zeros = tuple(jnp.zeros((_LANES,), jnp.float32) for _ in range(_NV))

    def block_body(blk, carry):
        row0 = gbase + blk * _BLK_GROUPS
        pltpu.sync_copy(ids_hbm.at[pl.ds(row0, _BLK_GROUPS), :], ids_v)
        for b in range(_NBUF):
            fire(b, b)

        def j_body(j4, inner):
            acc = zeros
            for b in range(_NBUF):
                g = j4 * _NBUF + b
                drain(g, b)
                init = zeros if b % 2 == 0 else acc
                acc = accum(bufs[b], init)

                @pl.when(j4 < (_BLK_GROUPS // _NBUF) - 1)
                def _():
                    fire(g + _NBUF, b)

                if b % 2 == 1:
                    row = blk * _ROWS_PER_BLK + j4 * 2 + b // 2
                    for c in range(_NV):
                        out_v[row, pl.ds(c * _LANES, _LANES)] = acc[c] * _SCALE
            return inner

        lax.fori_loop(0, _BLK_GROUPS // _NBUF, j_body, 0)
        return carry

    lax.fori_loop(0, _NBLK, block_body, 0)
    pltpu.sync_copy(out_v, out_hbm.at[pl.ds(wid * _ROWS_PER_W, _ROWS_PER_W), :])


_pool = functools.partial(
    pl.kernel,
    mesh=plsc.VectorSubcoreMesh(core_axis_name="c", subcore_axis_name="s"),
    out_type=jax.ShapeDtypeStruct((_B, _D), jnp.float32),
    scratch_types=[
        pltpu.VMEM((_BLK_GROUPS, _G), jnp.int32),
        pltpu.VMEM((_ROWS_PER_W, _D), jnp.float32),
        pltpu.VMEM((_G, _D), jnp.float32),
        pltpu.VMEM((_G, _D), jnp.float32),
        pltpu.VMEM((_G, _D), jnp.float32),
        pltpu.VMEM((_G, _D), jnp.float32),
        pltpu.SemaphoreType.DMA,
        pltpu.SemaphoreType.DMA,
        pltpu.SemaphoreType.DMA,
        pltpu.SemaphoreType.DMA,
    ],
)(_pool_body)


_VT = 512


def _mm_body(x_ref, w_ref, b_ref, o_ref):
    o_ref[...] = lax.dot_general(
        x_ref[...], w_ref[...], (((1,), (1,)), ((), ())),
        preferred_element_type=jnp.float32) + b_ref[...]


def _matmul(x, w, b2d):
    nv = pl.cdiv(_VOCAB, _VT)
    return pl.pallas_call(
        _mm_body,
        grid=(nv,),
        in_specs=[
            pl.BlockSpec((_B, _D), lambda j: (0, 0)),
            pl.BlockSpec((_VT, _D), lambda j: (j, 0)),
            pl.BlockSpec((1, _VT), lambda j: (0, j)),
        ],
        out_specs=pl.BlockSpec((_B, _VT), lambda j: (0, j)),
        out_shape=jax.ShapeDtypeStruct((_B, _VOCAB), jnp.float32),
    )(x, w, b2d)


def kernel(input_ids, emb_table, fc_w, fc_b):
    ids = input_ids.astype(jnp.int32).reshape(_B * _H // _G, _G)
    pooled = _pool(ids, emb_table)
    return _matmul(pooled, fc_w, fc_b.reshape(1, _VOCAB))


# speedup vs baseline: 1.6046x; 1.0043x over previous
import functools

import jax
import jax.numpy as jnp
from jax import lax
from jax.experimental import pallas as pl
from jax.experimental.pallas import tpu as pltpu
from jax.experimental.pallas import tpu_sc as plsc

_VOCAB = 100000
_D = 128
_B = 4096
_H = 200

_NC = 2
_NS = 16
_NW = _NC * _NS
_LANES = 16
_NV = _D // _LANES

_G = 100
_GROUPS_PER_ROW = _H // _G
_ROWS_PER_W = _B // _NW
_GROUPS_PER_W = _ROWS_PER_W * _GROUPS_PER_ROW
_BLK_GROUPS = 32
_NBLK = _GROUPS_PER_W // _BLK_GROUPS
_ROWS_PER_BLK = _BLK_GROUPS // _GROUPS_PER_ROW
_NBUF = 4
_SCALE = 1.0 / _H


def _pool_body(ids_hbm, table_hbm, out_hbm, ids_v, out_v,
               buf0, buf1, buf2, buf3, sem0, sem1, sem2, sem3):
    bufs = (buf0, buf1, buf2, buf3)
    sems = (sem0, sem1, sem2, sem3)
    wid = lax.axis_index("s") * _NC + lax.axis_index("c")
    gbase = wid * _GROUPS_PER_W

    def fire(g, b):
        pltpu.make_async_copy(table_hbm.at[ids_v.at[g]], bufs[b], sems[b]).start()

    def drain(g, b):
        pltpu.make_async_copy(table_hbm.at[ids_v.at[g]], bufs[b], sems[b]).wait()

    def accum(buf, init):
        def body(r, acc):
            return tuple(acc[c] + buf[r, pl.ds(c * _LANES, _LANES)]
                         for c in range(_NV))
        return lax.fori_loop(0, _G, body, init)

    zeros = tuple(jnp.zeros((_LANES,), jnp.float32) for _ in range(_NV))

    def block_body(blk, carry):
        row0 = gbase + blk * _BLK_GROUPS
        pltpu.sync_copy(ids_hbm.at[pl.ds(row0, _BLK_GROUPS), :], ids_v)
        for b in range(_NBUF):
            fire(b, b)

        def j_body(j4, inner):
            acc = zeros
            for b in range(_NBUF):
                g = j4 * _NBUF + b
                drain(g, b)
                init = zeros if b % 2 == 0 else acc
                acc = accum(bufs[b], init)

                @pl.when(j4 < (_BLK_GROUPS // _NBUF) - 1)
                def _():
                    fire(g + _NBUF, b)

                if b % 2 == 1:
                    row = blk * _ROWS_PER_BLK + j4 * 2 + b // 2
                    for c in range(_NV):
                        out_v[row, pl.ds(c * _LANES, _LANES)] = acc[c] * _SCALE
            return inner

        lax.fori_loop(0, _BLK_GROUPS // _NBUF, j_body, 0)
        return carry

    lax.fori_loop(0, _NBLK, block_body, 0)
    pltpu.sync_copy(out_v, out_hbm.at[pl.ds(wid * _ROWS_PER_W, _ROWS_PER_W), :])


_pool = functools.partial(
    pl.kernel,
    mesh=plsc.VectorSubcoreMesh(core_axis_name="c", subcore_axis_name="s"),
    out_type=jax.ShapeDtypeStruct((_B, _D), jnp.float32),
    scratch_types=[
        pltpu.VMEM((_BLK_GROUPS, _G), jnp.int32),
        pltpu.VMEM((_ROWS_PER_W, _D), jnp.float32),
        pltpu.VMEM((_G, _D), jnp.float32),
        pltpu.VMEM((_G, _D), jnp.float32),
        pltpu.VMEM((_G, _D), jnp.float32),
        pltpu.VMEM((_G, _D), jnp.float32),
        pltpu.SemaphoreType.DMA,
        pltpu.SemaphoreType.DMA,
        pltpu.SemaphoreType.DMA,
        pltpu.SemaphoreType.DMA,
    ],
)(_pool_body)


_VT = 1024


def _mm_body(x_ref, w_ref, b_ref, o_ref):
    o_ref[...] = lax.dot_general(
        x_ref[...], w_ref[...], (((1,), (1,)), ((), ())),
        preferred_element_type=jnp.float32) + b_ref[...]


def _matmul(x, w, b2d):
    nv = pl.cdiv(_VOCAB, _VT)
    return pl.pallas_call(
        _mm_body,
        grid=(nv,),
        in_specs=[
            pl.BlockSpec((_B, _D), lambda j: (0, 0)),
            pl.BlockSpec((_VT, _D), lambda j: (j, 0)),
            pl.BlockSpec((1, _VT), lambda j: (0, j)),
        ],
        out_specs=pl.BlockSpec((_B, _VT), lambda j: (0, j)),
        out_shape=jax.ShapeDtypeStruct((_B, _VOCAB), jnp.float32),
    )(x, w, b2d)


def kernel(input_ids, emb_table, fc_w, fc_b):
    ids = input_ids.astype(jnp.int32).reshape(_B * _H // _G, _G)
    pooled = _pool(ids, emb_table)
    return _matmul(pooled, fc_w, fc_b.reshape(1, _VOCAB))


# speedup vs baseline: 4.5898x; 2.8604x over previous
import functools

import jax
import jax.numpy as jnp
from jax import lax
from jax.experimental import pallas as pl
from jax.experimental.pallas import tpu as pltpu
from jax.experimental.pallas import tpu_sc as plsc

_VOCAB = 100000
_D = 128
_B = 4096
_H = 200

_NC = 2
_NS = 16
_NW = _NC * _NS
_LANES = 16
_NV = _D // _LANES

_G = 100
_GROUPS_PER_ROW = _H // _G
_ROWS_PER_W = _B // _NW
_GROUPS_PER_W = _ROWS_PER_W * _GROUPS_PER_ROW
_BLK_GROUPS = 32
_NBLK = _GROUPS_PER_W // _BLK_GROUPS
_ROWS_PER_BLK = _BLK_GROUPS // _GROUPS_PER_ROW
_NBUF = 4
_SCALE = 1.0 / _H


def _pool_body(ids_hbm, table_hbm, out_hbm, ids_v, out_v,
               buf0, buf1, buf2, buf3, sem0, sem1, sem2, sem3):
    bufs = (buf0, buf1, buf2, buf3)
    sems = (sem0, sem1, sem2, sem3)
    wid = lax.axis_index("s") * _NC + lax.axis_index("c")
    gbase = wid * _GROUPS_PER_W

    def fire(g, b):
        pltpu.make_async_copy(table_hbm.at[ids_v.at[g]], bufs[b], sems[b]).start()

    def drain(g, b):
        pltpu.make_async_copy(table_hbm.at[ids_v.at[g]], bufs[b], sems[b]).wait()

    def accum(buf, init):
        def body(r, acc):
            return tuple(acc[c] + buf[r, pl.ds(c * _LANES, _LANES)]
                         for c in range(_NV))
        return lax.fori_loop(0, _G, body, init)

    zeros = tuple(jnp.zeros((_LANES,), jnp.float32) for _ in range(_NV))

    def block_body(blk, carry):
        row0 = gbase + blk * _BLK_GROUPS
        pltpu.sync_copy(ids_hbm.at[pl.ds(row0, _BLK_GROUPS), :], ids_v)
        for b in range(_NBUF):
            fire(b, b)

        def j_body(j4, inner):
            acc = zeros
            for b in range(_NBUF):
                g = j4 * _NBUF + b
                drain(g, b)
                init = zeros if b % 2 == 0 else acc
                acc = accum(bufs[b], init)

                @pl.when(j4 < (_BLK_GROUPS // _NBUF) - 1)
                def _():
                    fire(g + _NBUF, b)

                if b % 2 == 1:
                    row = blk * _ROWS_PER_BLK + j4 * 2 + b // 2
                    for c in range(_NV):
                        out_v[row, pl.ds(c * _LANES, _LANES)] = acc[c] * _SCALE
            return inner

        lax.fori_loop(0, _BLK_GROUPS // _NBUF, j_body, 0)
        return carry

    lax.fori_loop(0, _NBLK, block_body, 0)
    pltpu.sync_copy(out_v, out_hbm.at[pl.ds(wid * _ROWS_PER_W, _ROWS_PER_W), :])


_pool = functools.partial(
    pl.kernel,
    mesh=plsc.VectorSubcoreMesh(core_axis_name="c", subcore_axis_name="s"),
    out_type=jax.ShapeDtypeStruct((_B, _D), jnp.float32),
    scratch_types=[
        pltpu.VMEM((_BLK_GROUPS, _G), jnp.int32),
        pltpu.VMEM((_ROWS_PER_W, _D), jnp.float32),
        pltpu.VMEM((_G, _D), jnp.float32),
        pltpu.VMEM((_G, _D), jnp.float32),
        pltpu.VMEM((_G, _D), jnp.float32),
        pltpu.VMEM((_G, _D), jnp.float32),
        pltpu.SemaphoreType.DMA,
        pltpu.SemaphoreType.DMA,
        pltpu.SemaphoreType.DMA,
        pltpu.SemaphoreType.DMA,
    ],
)(_pool_body)


_VT = 512


def _mm_body(w_ref, x_ref, b_ref, o_ref):
    o_ref[...] = lax.dot_general(
        w_ref[...], x_ref[...], (((1,), (1,)), ((), ())),
        preferred_element_type=jnp.float32) + b_ref[...]


def _matmul_t(x, w, bcol):
    nv = pl.cdiv(_VOCAB, _VT)
    return pl.pallas_call(
        _mm_body,
        grid=(nv,),
        in_specs=[
            pl.BlockSpec((_VT, _D), lambda j: (j, 0)),
            pl.BlockSpec((_B, _D), lambda j: (0, 0)),
            pl.BlockSpec((_VT, 1), lambda j: (j, 0)),
        ],
        out_specs=pl.BlockSpec((_VT, _B), lambda j: (j, 0)),
        out_shape=jax.ShapeDtypeStruct((_VOCAB, _B), jnp.float32),
    )(w, x, bcol)


def kernel(input_ids, emb_table, fc_w, fc_b):
    ids = input_ids.astype(jnp.int32).reshape(_B * _H // _G, _G)
    pooled = _pool(ids, emb_table)
    logits_t = _matmul_t(pooled, fc_w, fc_b.reshape(_VOCAB, 1))
    return logits_t.T
